# Initial kernel scaffold; baseline (speedup 1.0000x reference)
import functools

import jax
import jax.numpy as jnp
from jax import lax
from jax.experimental import pallas as pl
from jax.experimental.pallas import tpu as pltpu
from jax.experimental.pallas import tpu_sc as plsc

_NC = 2
_NS = 16
_NW = _NC * _NS
_LANES = 16
_CHUNK = 128


def _sc_gather(scores_flat, idx2d, n_cols, k_per_row):
    n_chunks_total = idx2d.shape[0]
    assert n_chunks_total % _NW == 0
    n_chunks = n_chunks_total // _NW
    vec_per_chunk = _CHUNK // _LANES

    mesh = plsc.VectorSubcoreMesh(core_axis_name="c", subcore_axis_name="s")

    @functools.partial(
        pl.kernel,
        out_type=jax.ShapeDtypeStruct((n_chunks_total, _CHUNK), jnp.float32),
        mesh=mesh,
        scratch_types=[
            pltpu.VMEM((n_chunks, _CHUNK), jnp.int32),
            pltpu.VMEM((n_chunks, _CHUNK), jnp.float32),
            pltpu.SemaphoreType.DMA,
        ],
    )
    def gather_kernel(scores_hbm, idx_hbm, out_hbm, idx_v, vals_v, sem):
        wid = lax.axis_index("s") * _NC + lax.axis_index("c")
        base = wid * n_chunks
        pltpu.sync_copy(idx_hbm.at[pl.ds(base, n_chunks)], idx_v)

        elem_base = base * _CHUNK
        lane = lax.iota(jnp.int32, _LANES)

        def offs_body(c, _):
            j = c // vec_per_chunk
            col = (c % vec_per_chunk) * _LANES
            q = elem_base + c * _LANES + lane
            row = q // k_per_row
            cur = idx_v[j, pl.ds(col, _LANES)]
            idx_v[j, pl.ds(col, _LANES)] = cur + row * n_cols
            return 0

        lax.fori_loop(0, n_chunks * vec_per_chunk, offs_body, 0)

        def gat_body(j, _):
            cp = pltpu.make_async_copy(
                scores_hbm.at[idx_v.at[j]], vals_v.at[j], sem
            )
            cp.start()
            cp.wait()
            return 0

        lax.fori_loop(0, n_chunks, gat_body, 0)

        pltpu.sync_copy(vals_v, out_hbm.at[pl.ds(base, n_chunks)])

    return gather_kernel(scores_flat, idx2d)


def _tc_log_softmax(g):
    def body(x_ref, o_ref):
        x = x_ref[...]
        m = jnp.max(x, axis=1, keepdims=True)
        e = jnp.exp(x - m)
        s = jnp.sum(e, axis=1, keepdims=True)
        o_ref[...] = (x - m) - jnp.log(s)

    return pl.pallas_call(
        body,
        out_shape=jax.ShapeDtypeStruct(g.shape, g.dtype),
    )(g)


def kernel(scores, idx):
    b, n = scores.shape
    k = idx.shape[1]
    bk = b * k
    assert bk % (_CHUNK * _NW) == 0
    gathered = _sc_gather(
        scores.reshape(-1), idx.reshape(bk // _CHUNK, _CHUNK), n, k
    )
    return _tc_log_softmax(gathered.reshape(b, k))


# speedup vs baseline: 2.0533x; 2.0533x over previous
import functools

import jax
import jax.numpy as jnp
from jax import lax
from jax.experimental import pallas as pl
from jax.experimental.pallas import tpu as pltpu
from jax.experimental.pallas import tpu_sc as plsc

_NC = 2
_NS = 16
_NW = _NC * _NS
_LANES = 16
_CHUNK = 128


def _sc_gather(scores_flat, idx2d, n_cols, k_per_row):
    n_chunks_total = idx2d.shape[0]
    assert n_chunks_total % _NW == 0
    n_chunks = n_chunks_total // _NW
    vec_per_chunk = _CHUNK // _LANES
    idx3d = idx2d.reshape(_NW, n_chunks, _CHUNK)

    mesh = plsc.VectorSubcoreMesh(core_axis_name="c", subcore_axis_name="s")

    @functools.partial(
        pl.kernel,
        out_type=jax.ShapeDtypeStruct((_NW, n_chunks, _CHUNK), jnp.float32),
        mesh=mesh,
        scratch_types=[
            pltpu.VMEM((n_chunks, _CHUNK), jnp.int32),
            pltpu.VMEM((n_chunks, _CHUNK), jnp.float32),
            pltpu.SemaphoreType.DMA,
        ],
    )
    def gather_kernel(scores_hbm, idx_hbm, out_hbm, idx_v, vals_v, sem):
        wid = lax.axis_index("s") * _NC + lax.axis_index("c")
        pltpu.sync_copy(idx_hbm.at[wid], idx_v)

        elem_base = wid * (n_chunks * _CHUNK)
        lane = lax.iota(jnp.int32, _LANES)

        def offs_body(c, _):
            j = c // vec_per_chunk
            col = (c % vec_per_chunk) * _LANES
            q0 = elem_base + c * _LANES
            q = lax.broadcast(q0, (_LANES,)) + lane
            row = lax.div(q, jnp.int32(k_per_row))
            cur = idx_v[j, pl.ds(col, _LANES)]
            idx_v[j, pl.ds(col, _LANES)] = cur + row * jnp.int32(n_cols)
            return 0

        lax.fori_loop(0, n_chunks * vec_per_chunk, offs_body, 0)

        def gat_body(j, _):
            cp = pltpu.make_async_copy(
                scores_hbm.at[idx_v.at[j]], vals_v.at[j], sem
            )
            cp.start()
            cp.wait()
            return 0

        lax.fori_loop(0, n_chunks, gat_body, 0)

        pltpu.sync_copy(vals_v, out_hbm.at[wid])

    return gather_kernel(scores_flat, idx3d)


def _tc_log_softmax(g):
    def body(x_ref, o_ref):
        x = x_ref[...]
        m = jnp.max(x, axis=1, keepdims=True)
        e = jnp.exp(x - m)
        s = jnp.sum(e, axis=1, keepdims=True)
        o_ref[...] = (x - m) - jnp.log(s)

    return pl.pallas_call(
        body,
        out_shape=jax.ShapeDtypeStruct(g.shape, g.dtype),
    )(g)


def kernel(scores, idx):
    b, n = scores.shape
    k = idx.shape[1]
    bk = b * k
    assert bk % (_CHUNK * _NW) == 0
    gathered = _sc_gather(
        scores.reshape(-1), idx.reshape(bk // _CHUNK, _CHUNK), n, k
    )
    return _tc_log_softmax(gathered.reshape(b, k))


# speedup vs baseline: 26.9113x; 13.1062x over previous
import functools

import jax
import jax.numpy as jnp
from jax import lax
from jax.experimental import pallas as pl
from jax.experimental.pallas import tpu as pltpu
from jax.experimental.pallas import tpu_sc as plsc

_NC = 2
_NS = 16
_NW = _NC * _NS
_LANES = 16
_CHUNK = 128


def _sc_gather(scores_flat, idx2d, n_bt, k_per_row):
    n_chunks_total = idx2d.shape[0]
    assert n_chunks_total % _NW == 0
    n_chunks = n_chunks_total // _NW
    vec_per_chunk = _CHUNK // _LANES
    idx3d = idx2d.reshape(_NW, n_chunks, _CHUNK)

    mesh = plsc.VectorSubcoreMesh(core_axis_name="c", subcore_axis_name="s")

    @functools.partial(
        pl.kernel,
        out_type=jax.ShapeDtypeStruct((_NW, n_chunks, _CHUNK), jnp.float32),
        mesh=mesh,
        scratch_types=[
            pltpu.VMEM((n_chunks, _CHUNK), jnp.int32),
            pltpu.VMEM((n_chunks, _CHUNK), jnp.float32),
            pltpu.SemaphoreType.DMA,
        ],
    )
    def gather_kernel(scores_hbm, idx_hbm, out_hbm, idx_v, vals_v, sem):
        wid = lax.axis_index("s") * _NC + lax.axis_index("c")
        pltpu.sync_copy(idx_hbm.at[wid], idx_v)

        elem_base = wid * (n_chunks * _CHUNK)
        lane = lax.iota(jnp.int32, _LANES)

        def offs_body(c, _):
            j = c // vec_per_chunk
            col = (c % vec_per_chunk) * _LANES
            q0 = elem_base + c * _LANES
            q = lax.broadcast(q0, (_LANES,)) + lane
            row = lax.div(q, jnp.int32(k_per_row))
            cur = idx_v[j, pl.ds(col, _LANES)]
            jc = lax.shift_right_logical(cur, 3)
            jr = lax.bitwise_and(cur, jnp.int32(7))
            bt = lax.shift_right_logical(row, 7)
            bc = lax.bitwise_and(row, jnp.int32(127))
            flat = (
                jc * jnp.int32(n_bt * 1024)
                + lax.shift_left(bt, 10)
                + lax.shift_left(jr, 7)
                + bc
            )
            idx_v[j, pl.ds(col, _LANES)] = flat
            return 0

        lax.fori_loop(0, n_chunks * vec_per_chunk, offs_body, 0)

        def gat_body(j, _):
            cp = pltpu.make_async_copy(
                scores_hbm.at[idx_v.at[j]], vals_v.at[j], sem
            )
            cp.start()
            cp.wait()
            return 0

        lax.fori_loop(0, n_chunks, gat_body, 0)

        pltpu.sync_copy(vals_v, out_hbm.at[wid])

    return gather_kernel(scores_flat, idx3d)


def _tc_log_softmax(g):
    def body(x_ref, o_ref):
        x = x_ref[...]
        m = jnp.max(x, axis=1, keepdims=True)
        e = jnp.exp(x - m)
        s = jnp.sum(e, axis=1, keepdims=True)
        o_ref[...] = (x - m) - jnp.log(s)

    return pl.pallas_call(
        body,
        out_shape=jax.ShapeDtypeStruct(g.shape, g.dtype),
    )(g)


def kernel(scores, idx):
    b, n = scores.shape
    k = idx.shape[1]
    bk = b * k
    assert bk % (_CHUNK * _NW) == 0
    assert b % 128 == 0 and n % 8 == 0
    scores_perm = (
        scores.reshape(b // 128, 128, n // 8, 8)
        .transpose(2, 0, 3, 1)
        .reshape(-1)
    )
    gathered = _sc_gather(
        scores_perm, idx.reshape(bk // _CHUNK, _CHUNK), b // 128, k
    )
    return _tc_log_softmax(gathered.reshape(b, k))


# speedup vs baseline: 57.1844x; 2.1249x over previous
import functools

import jax
import jax.numpy as jnp
from jax import lax
from jax.experimental import pallas as pl
from jax.experimental.pallas import tpu as pltpu
from jax.experimental.pallas import tpu_sc as plsc

_NC = 2
_NS = 16
_NW = _NC * _NS
_LANES = 16
_CHUNK = 128


def _perm_flat(x):
    b, m = x.shape
    return x.reshape(b // 128, 128, m // 8, 8).transpose(2, 0, 3, 1).reshape(-1)


def _sc_gather(scores_flat, idx_flat, n_bt):
    total = idx_flat.shape[0]
    assert total % (_NW * _CHUNK) == 0
    per_w = total // _NW
    n_chunks = per_w // _CHUNK
    n_vec = per_w // _LANES

    mesh = plsc.VectorSubcoreMesh(core_axis_name="c", subcore_axis_name="s")

    @functools.partial(
        pl.kernel,
        out_type=jax.ShapeDtypeStruct((total,), jnp.float32),
        mesh=mesh,
        scratch_types=[
            pltpu.VMEM((per_w,), jnp.int32),
            pltpu.VMEM((per_w,), jnp.float32),
            pltpu.SemaphoreType.DMA,
        ],
    )
    def gather_kernel(scores_hbm, idx_hbm, out_hbm, idx_v, vals_v, sem):
        wid = lax.axis_index("s") * _NC + lax.axis_index("c")
        elem_base = wid * per_w
        pltpu.sync_copy(idx_hbm.at[pl.ds(elem_base, per_w)], idx_v)

        lane = lax.iota(jnp.int32, _LANES)

        def offs_body(c, _):
            col = c * _LANES
            q = lax.broadcast(elem_base + col, (_LANES,)) + lane
            bt = lax.bitwise_and(
                lax.shift_right_logical(q, 10), jnp.int32(n_bt - 1)
            )
            bc = lax.bitwise_and(q, jnp.int32(127))
            j = idx_v[pl.ds(col, _LANES)]
            jc = lax.shift_right_logical(j, 3)
            jr = lax.bitwise_and(j, jnp.int32(7))
            idx_v[pl.ds(col, _LANES)] = (
                jc * jnp.int32(n_bt * 1024)
                + lax.shift_left(bt, 10)
                + lax.shift_left(jr, 7)
                + bc
            )
            return 0

        lax.fori_loop(0, n_vec, offs_body, 0)

        def fire(c, _):
            pltpu.make_async_copy(
                scores_hbm.at[idx_v.at[pl.ds(c * _CHUNK, _CHUNK)]],
                vals_v.at[pl.ds(c * _CHUNK, _CHUNK)],
                sem,
            ).start()
            return 0

        def drain(c, _):
            pltpu.make_async_copy(
                scores_hbm.at[idx_v.at[pl.ds(0, _CHUNK)]],
                vals_v.at[pl.ds(0, _CHUNK)],
                sem,
            ).wait()
            return 0

        lax.fori_loop(0, n_chunks, fire, 0)
        lax.fori_loop(0, n_chunks, drain, 0)

        pltpu.sync_copy(vals_v, out_hbm.at[pl.ds(elem_base, per_w)])

    return gather_kernel(scores_flat, idx_flat)


def _tc_log_softmax_perm(g4):
    def body(x_ref, o_ref):
        x = x_ref[...]
        m = jnp.max(jnp.max(x, axis=0, keepdims=True), axis=2, keepdims=True)
        e = jnp.exp(x - m)
        s = jnp.sum(jnp.sum(e, axis=0, keepdims=True), axis=2, keepdims=True)
        o_ref[...] = (x - m) - jnp.log(s)

    return pl.pallas_call(
        body,
        out_shape=jax.ShapeDtypeStruct(g4.shape, g4.dtype),
    )(g4)


def kernel(scores, idx):
    b, n = scores.shape
    k = idx.shape[1]
    assert b % 128 == 0 and n % 8 == 0 and k % 8 == 0
    assert (b * k) % (_NW * _CHUNK) == 0
    n_bt = b // 128
    assert n_bt & (n_bt - 1) == 0

    gathered = _sc_gather(_perm_flat(scores), _perm_flat(idx), n_bt)
    out4 = _tc_log_softmax_perm(gathered.reshape(k // 8, n_bt, 8, 128))
    return out4.transpose(1, 3, 0, 2).reshape(b, k)


# speedup vs baseline: 61.3235x; 1.0724x over previous
import functools

import jax
import jax.numpy as jnp
from jax import lax
from jax.experimental import pallas as pl
from jax.experimental.pallas import tpu as pltpu
from jax.experimental.pallas import tpu_sc as plsc

_NC = 2
_NS = 16
_NW = _NC * _NS
_LANES = 16
_CHUNK = 128


def _perm_flat(x):
    b, m = x.shape
    return x.reshape(b // 128, 128, m // 8, 8).transpose(2, 0, 3, 1).reshape(-1)


def _sc_gather(scores_flat, idx_flat, n_bt):
    total = idx_flat.shape[0]
    assert total % (_NW * _CHUNK) == 0
    per_w = total // _NW
    n_chunks = per_w // _CHUNK
    n_vec = per_w // _LANES

    mesh = plsc.VectorSubcoreMesh(core_axis_name="c", subcore_axis_name="s")

    @functools.partial(
        pl.kernel,
        out_type=jax.ShapeDtypeStruct((total,), jnp.float32),
        mesh=mesh,
        scratch_types=[
            pltpu.VMEM((per_w,), jnp.int32),
            pltpu.VMEM((per_w,), jnp.float32),
            pltpu.SemaphoreType.DMA,
        ],
    )
    def gather_kernel(scores_hbm, idx_hbm, out_hbm, idx_v, vals_v, sem):
        wid = lax.axis_index("s") * _NC + lax.axis_index("c")
        elem_base = wid * per_w
        pltpu.sync_copy(idx_hbm.at[pl.ds(elem_base, per_w)], idx_v)

        lane = lax.iota(jnp.int32, _LANES)
        sh = 10 + (n_bt.bit_length() - 1)
        m_hi = jnp.int32(-(1 << sh))
        m_jr = jnp.int32(7 << 7)
        m_b = jnp.int32(((n_bt - 1) << 10) | 127)

        def chunk_body(c, _):
            col0 = c * _CHUNK
            for v in range(_CHUNK // _LANES):
                col = col0 + v * _LANES
                q = lax.broadcast(elem_base + col, (_LANES,)) + lane
                j = idx_v[pl.ds(col, _LANES)]
                hi = lax.bitwise_and(lax.shift_left(j, sh - 3), m_hi)
                jr = lax.bitwise_and(lax.shift_left(j, 7), m_jr)
                idx_v[pl.ds(col, _LANES)] = lax.bitwise_or(
                    lax.bitwise_or(hi, jr), lax.bitwise_and(q, m_b)
                )
            pltpu.make_async_copy(
                scores_hbm.at[idx_v.at[pl.ds(col0, _CHUNK)]],
                vals_v.at[pl.ds(col0, _CHUNK)],
                sem,
            ).start()
            return 0

        def drain(c, _):
            pltpu.make_async_copy(
                scores_hbm.at[idx_v.at[pl.ds(0, _CHUNK)]],
                vals_v.at[pl.ds(0, _CHUNK)],
                sem,
            ).wait()
            return 0

        lax.fori_loop(0, n_chunks, chunk_body, 0)
        lax.fori_loop(0, n_chunks, drain, 0)

        pltpu.sync_copy(vals_v, out_hbm.at[pl.ds(elem_base, per_w)])

    return gather_kernel(scores_flat, idx_flat)


def _tc_log_softmax_perm(g4):
    def body(x_ref, o_ref):
        x = x_ref[...]
        m = jnp.max(jnp.max(x, axis=0, keepdims=True), axis=2, keepdims=True)
        e = jnp.exp(x - m)
        s = jnp.sum(jnp.sum(e, axis=0, keepdims=True), axis=2, keepdims=True)
        o_ref[...] = (x - m) - jnp.log(s)

    return pl.pallas_call(
        body,
        out_shape=jax.ShapeDtypeStruct(g4.shape, g4.dtype),
    )(g4)


def kernel(scores, idx):
    b, n = scores.shape
    k = idx.shape[1]
    assert b % 128 == 0 and n % 8 == 0 and k % 8 == 0
    assert (b * k) % (_NW * _CHUNK) == 0
    n_bt = b // 128
    assert n_bt & (n_bt - 1) == 0

    gathered = _sc_gather(_perm_flat(scores), _perm_flat(idx), n_bt)
    out4 = _tc_log_softmax_perm(gathered.reshape(k // 8, n_bt, 8, 128))
    return out4.transpose(1, 3, 0, 2).reshape(b, k)
